# baseline (speedup 1.0000x reference)
import jax
import jax.numpy as jnp
from jax import lax
from jax.experimental import pallas as pl
from jax.experimental.pallas import tpu as pltpu
from jax.experimental.pallas import tpu_sc as plsc

B = 64
C = 768
H = 24
ROWS = B * C

NC = 2
NS = 16
NW = NC * NS
ROWS_PER_W = ROWS // NW
BATCHES_PER_W = ROWS_PER_W // C
CHUNK = 32
NCHUNK = ROWS_PER_W // CHUNK
CHUNKS_PER_BATCH = C // CHUNK
NBUF = 4
LEAD = 2


def _sc_shuffle(x_hbm, idx_hbm, out_hbm, idx_v, gidx_v,
                buf0, buf1, buf2, buf3,
                sem_g0, sem_g1, sem_g2, sem_g3,
                sem_o0, sem_o1, sem_o2, sem_o3):
    cid = lax.axis_index("c")
    sid = lax.axis_index("s")
    wid = sid * NC + cid
    base_row = wid * ROWS_PER_W
    first_batch = wid * BATCHES_PER_W

    pltpu.sync_copy(idx_hbm, idx_v)

    for ci in range(NCHUNK):
        h = ci // CHUNKS_PER_BATCH
        j0 = (ci % CHUNKS_PER_BATCH) * CHUNK
        row_base = (first_batch + h) * C
        for ki in range(CHUNK // 16):
            vals = idx_v[pl.ds(j0 + 16 * ki, 16)] + row_base
            gidx_v[ci, pl.ds(16 * ki, 16)] = vals

    bufs = (buf0, buf1, buf2, buf3)
    gsems = (sem_g0, sem_g1, sem_g2, sem_g3)
    osems = (sem_o0, sem_o1, sem_o2, sem_o3)
    gathers = [None] * NBUF
    outs = [None] * NBUF

    for ci in range(LEAD):
        gathers[ci % NBUF] = pltpu.async_copy(
            x_hbm.at[gidx_v.at[ci]], bufs[ci % NBUF], gsems[ci % NBUF])

    for ci in range(NCHUNK):
        b = ci % NBUF
        gathers[b].wait()
        outs[b] = pltpu.async_copy(
            bufs[b], out_hbm.at[pl.ds(base_row + ci * CHUNK, CHUNK)],
            osems[b])
        nxt = ci + LEAD
        if nxt < NCHUNK:
            bn = nxt % NBUF
            if outs[bn] is not None:
                outs[bn].wait()
                outs[bn] = None
            gathers[bn] = pltpu.async_copy(
                x_hbm.at[gidx_v.at[nxt]], bufs[bn], gsems[bn])

    for b in range(NBUF):
        if outs[b] is not None:
            outs[b].wait()


@jax.jit
def _shuffle(x, forward_shuffle_idx):
    xr = x.reshape(ROWS, H, H)
    mesh = plsc.VectorSubcoreMesh(core_axis_name="c", subcore_axis_name="s")
    run = pl.kernel(
        _sc_shuffle,
        out_type=jax.ShapeDtypeStruct((ROWS, H, H), jnp.float32),
        mesh=mesh,
        scratch_types=[
            pltpu.VMEM((C,), jnp.int32),
            pltpu.VMEM((NCHUNK, CHUNK), jnp.int32),
            pltpu.VMEM((CHUNK, H, H), jnp.float32),
            pltpu.VMEM((CHUNK, H, H), jnp.float32),
            pltpu.VMEM((CHUNK, H, H), jnp.float32),
            pltpu.VMEM((CHUNK, H, H), jnp.float32),
            pltpu.SemaphoreType.DMA,
            pltpu.SemaphoreType.DMA,
            pltpu.SemaphoreType.DMA,
            pltpu.SemaphoreType.DMA,
            pltpu.SemaphoreType.DMA,
            pltpu.SemaphoreType.DMA,
            pltpu.SemaphoreType.DMA,
            pltpu.SemaphoreType.DMA,
        ],
        compiler_params=pltpu.CompilerParams(use_tc_tiling_on_sc=False),
    )
    out = run(xr, forward_shuffle_idx)
    return out.reshape(B, C, H, H)


def kernel(x, forward_shuffle_idx):
    return (_shuffle(x, forward_shuffle_idx), 0)


# speedup vs baseline: 3.2515x; 3.2515x over previous
import jax
import jax.numpy as jnp
from jax import lax
from jax.experimental import pallas as pl
from jax.experimental.pallas import tpu as pltpu
from jax.experimental.pallas import tpu_sc as plsc

B = 64
C = 768
H = 24
RTILES = B * H * H // 8
WORDS_RT = 8 * C
NWORDS = RTILES * WORDS_RT

NC = 2
NS = 16
NW = NC * NS
RT_PER_W = RTILES // NW
SLAB_RT = 4
SLAB_W = SLAB_RT * WORDS_RT
NSLAB = RT_PER_W // SLAB_RT


def _permute_slab(in_buf, out_buf, poff_v):
    def row_body(r, carry):
        rt = r >> 3
        s = r & 7
        base = rt * WORDS_RT + s * 128
        for k in range(C // 16):
            src = poff_v[pl.ds(16 * k, 16)] + base
            val = plsc.load_gather(in_buf, [src])
            out_off = base + (k // 8) * 1024 + (k % 8) * 16
            out_buf[pl.ds(out_off, 16)] = val
        return carry

    lax.fori_loop(0, SLAB_RT * 8, row_body, 0)


def _sc_shuffle(x_hbm, idx_hbm, out_hbm, idx_v, poff_v,
                in_a, in_b, out_a, out_b,
                sem_ia, sem_ib, sem_oa, sem_ob):
    cid = lax.axis_index("c")
    sid = lax.axis_index("s")
    wid = sid * NC + cid
    base_w = wid * RT_PER_W * WORDS_RT

    pltpu.sync_copy(idx_hbm, idx_v)
    for k in range(C // 16):
        v = idx_v[pl.ds(16 * k, 16)]
        poff_v[pl.ds(16 * k, 16)] = ((v >> 7) << 10) + (v & 127)

    ins = (in_a, in_b)
    outs = (out_a, out_b)
    isems = (sem_ia, sem_ib)
    osems = (sem_oa, sem_ob)

    for ph in range(2):
        pltpu.async_copy(x_hbm.at[pl.ds(base_w + ph * SLAB_W, SLAB_W)],
                         ins[ph], isems[ph])

    def pair(p, carry):
        for ph in range(2):
            s_idx = p * 2 + ph
            off = base_w + s_idx * SLAB_W

            pltpu.make_async_copy(
                x_hbm.at[pl.ds(off, SLAB_W)], ins[ph], isems[ph]).wait()

            @pl.when(p > 0)
            def _():
                pltpu.make_async_copy(
                    outs[ph],
                    out_hbm.at[pl.ds(off - 2 * SLAB_W, SLAB_W)],
                    osems[ph]).wait()

            _permute_slab(ins[ph], outs[ph], poff_v)

            @pl.when(s_idx + 2 < NSLAB)
            def _():
                pltpu.async_copy(
                    x_hbm.at[pl.ds(off + 2 * SLAB_W, SLAB_W)],
                    ins[ph], isems[ph])

            pltpu.async_copy(
                outs[ph], out_hbm.at[pl.ds(off, SLAB_W)], osems[ph])
        return carry

    lax.fori_loop(0, NSLAB // 2, pair, 0)

    for ph in range(2):
        last = base_w + (NSLAB - 2 + ph) * SLAB_W
        pltpu.make_async_copy(
            outs[ph], out_hbm.at[pl.ds(last, SLAB_W)], osems[ph]).wait()


@jax.jit
def _shuffle(x, forward_shuffle_idx):
    xv = (x.transpose(0, 2, 3, 1)
           .reshape(RTILES, 8, C // 128, 128)
           .transpose(0, 2, 1, 3)
           .reshape(NWORDS))
    mesh = plsc.VectorSubcoreMesh(core_axis_name="c", subcore_axis_name="s")
    run = pl.kernel(
        _sc_shuffle,
        out_type=jax.ShapeDtypeStruct((NWORDS,), jnp.float32),
        mesh=mesh,
        scratch_types=[
            pltpu.VMEM((C,), jnp.int32),
            pltpu.VMEM((C,), jnp.int32),
            pltpu.VMEM((SLAB_W,), jnp.float32),
            pltpu.VMEM((SLAB_W,), jnp.float32),
            pltpu.VMEM((SLAB_W,), jnp.float32),
            pltpu.VMEM((SLAB_W,), jnp.float32),
            pltpu.SemaphoreType.DMA,
            pltpu.SemaphoreType.DMA,
            pltpu.SemaphoreType.DMA,
            pltpu.SemaphoreType.DMA,
        ],
        compiler_params=pltpu.CompilerParams(use_tc_tiling_on_sc=False, needs_layout_passes=False),
    )
    out = run(xv, forward_shuffle_idx)
    out = (out.reshape(RTILES, C // 128, 8, 128)
              .transpose(0, 2, 1, 3)
              .reshape(B, H, H, C)
              .transpose(0, 3, 1, 2))
    return out


def kernel(x, forward_shuffle_idx):
    return (_shuffle(x, forward_shuffle_idx), 0)


# speedup vs baseline: 11.9418x; 3.6727x over previous
import jax
import jax.numpy as jnp
from jax import lax
from jax.experimental import pallas as pl
from jax.experimental.pallas import tpu as pltpu
from jax.experimental.pallas import tpu_sc as plsc

B = 64
C = 768
H = 24
RTILES = B * H * H // 8
WORDS_RT = 8 * C
NWORDS = RTILES * WORDS_RT

NC = 2
NS = 16
NW = NC * NS
RT_PER_W = RTILES // NW
SLAB_RT = 4
SLAB_W = SLAB_RT * WORDS_RT
NSLAB = RT_PER_W // SLAB_RT


def _permute_slab(in_buf, out_buf, poff_v):
    @plsc.parallel_loop(0, SLAB_RT * 8, unroll=2)
    def row_body(r):
        rt = r >> 3
        s = r & 7
        base = rt * WORDS_RT + s * 128
        for k in range(C // 16):
            src = poff_v[pl.ds(16 * k, 16)] + base
            val = plsc.load_gather(in_buf, [src])
            out_off = base + (k // 8) * 1024 + (k % 8) * 16
            out_buf[pl.ds(out_off, 16)] = val


def _sc_shuffle(x_hbm, idx_hbm, out_hbm, idx_v, poff_v,
                in_a, in_b, out_a, out_b,
                sem_ia, sem_ib, sem_oa, sem_ob):
    cid = lax.axis_index("c")
    sid = lax.axis_index("s")
    wid = sid * NC + cid
    base_w = wid * RT_PER_W * WORDS_RT

    pltpu.sync_copy(idx_hbm, idx_v)
    for k in range(C // 16):
        v = idx_v[pl.ds(16 * k, 16)]
        poff_v[pl.ds(16 * k, 16)] = ((v >> 7) << 10) + (v & 127)

    ins = (in_a, in_b)
    outs = (out_a, out_b)
    isems = (sem_ia, sem_ib)
    osems = (sem_oa, sem_ob)

    for ph in range(2):
        pltpu.async_copy(x_hbm.at[pl.ds(base_w + ph * SLAB_W, SLAB_W)],
                         ins[ph], isems[ph])

    def pair(p, carry):
        for ph in range(2):
            s_idx = p * 2 + ph
            off = base_w + s_idx * SLAB_W

            pltpu.make_async_copy(
                x_hbm.at[pl.ds(off, SLAB_W)], ins[ph], isems[ph]).wait()

            @pl.when(p > 0)
            def _():
                pltpu.make_async_copy(
                    outs[ph],
                    out_hbm.at[pl.ds(off - 2 * SLAB_W, SLAB_W)],
                    osems[ph]).wait()

            _permute_slab(ins[ph], outs[ph], poff_v)

            @pl.when(s_idx + 2 < NSLAB)
            def _():
                pltpu.async_copy(
                    x_hbm.at[pl.ds(off + 2 * SLAB_W, SLAB_W)],
                    ins[ph], isems[ph])

            pltpu.async_copy(
                outs[ph], out_hbm.at[pl.ds(off, SLAB_W)], osems[ph])
        return carry

    lax.fori_loop(0, NSLAB // 2, pair, 0)

    for ph in range(2):
        last = base_w + (NSLAB - 2 + ph) * SLAB_W
        pltpu.make_async_copy(
            outs[ph], out_hbm.at[pl.ds(last, SLAB_W)], osems[ph]).wait()


@jax.jit
def _shuffle(x, forward_shuffle_idx):
    xv = (x.transpose(0, 2, 3, 1)
           .reshape(RTILES, 8, C // 128, 128)
           .transpose(0, 2, 1, 3)
           .reshape(NWORDS))
    mesh = plsc.VectorSubcoreMesh(core_axis_name="c", subcore_axis_name="s")
    run = pl.kernel(
        _sc_shuffle,
        out_type=jax.ShapeDtypeStruct((NWORDS,), jnp.float32),
        mesh=mesh,
        scratch_types=[
            pltpu.VMEM((C,), jnp.int32),
            pltpu.VMEM((C,), jnp.int32),
            pltpu.VMEM((SLAB_W,), jnp.float32),
            pltpu.VMEM((SLAB_W,), jnp.float32),
            pltpu.VMEM((SLAB_W,), jnp.float32),
            pltpu.VMEM((SLAB_W,), jnp.float32),
            pltpu.SemaphoreType.DMA,
            pltpu.SemaphoreType.DMA,
            pltpu.SemaphoreType.DMA,
            pltpu.SemaphoreType.DMA,
        ],
        compiler_params=pltpu.CompilerParams(use_tc_tiling_on_sc=False, needs_layout_passes=False),
    )
    out = run(xv, forward_shuffle_idx)
    out = (out.reshape(RTILES, C // 128, 8, 128)
              .transpose(0, 2, 1, 3)
              .reshape(B, H, H, C)
              .transpose(0, 3, 1, 2))
    return out


def kernel(x, forward_shuffle_idx):
    return (_shuffle(x, forward_shuffle_idx), 0)
